# baseline (device time: 426980 ns/iter reference)
import jax
import jax.numpy as jnp
from jax import lax
from jax.experimental import pallas as pl
from jax.experimental.pallas import tpu as pltpu

N_DEV = 16
E_LOC = 4


def kernel(x, router_W, route_idx, expert_W):
    T, D = x.shape
    E = router_W.shape[1]
    H = expert_W.shape[2]

    def body(x_ref, rw_ref, idx_ref, ew_ref, out_ref,
             comm_ref, send_sems, recv_sems, credit_sem):
        my = lax.axis_index("i")
        left = lax.rem(my - 1 + N_DEV, N_DEV)
        right = lax.rem(my + 1, N_DEV)

        barrier_sem = pltpu.get_barrier_semaphore()
        for nbr in (left, right):
            pl.semaphore_signal(
                barrier_sem, inc=1,
                device_id=(nbr,), device_id_type=pl.DeviceIdType.MESH,
            )
        pl.semaphore_wait(barrier_sem, 2)

        xv = x_ref[:, :]
        scores = jnp.dot(xv, rw_ref[:, :], preferred_element_type=jnp.float32)
        e0 = idx_ref[:, 0:1]
        e1 = idx_ref[:, 1:2]
        cols = lax.broadcasted_iota(jnp.int32, (T, E), 1)
        s0 = jnp.sum(jnp.where(cols == e0, scores, 0.0), axis=1, keepdims=True)
        s1 = jnp.sum(jnp.where(cols == e1, scores, 0.0), axis=1, keepdims=True)
        w0 = 1.0 / (1.0 + jnp.exp(s1 - s0))
        w1 = 1.0 - w0

        def accumulate(origin, w_block, init):
            contrib = None
            for k in range(E_LOC):
                e_id = origin * E_LOC + k
                col = (jnp.where(e0 == e_id, w0, 0.0)
                       + jnp.where(e1 == e_id, w1, 0.0))
                part = jnp.dot(xv * col, w_block[k],
                               preferred_element_type=jnp.float32)
                contrib = part if contrib is None else contrib + part
            if init:
                out_ref[:, :] = contrib
            else:
                out_ref[:, :] += contrib

        accumulate(my, ew_ref[:, :, :], init=True)

        for j in range(N_DEV - 1):
            src = ew_ref if j == 0 else comm_ref.at[j % 2]
            rdma = pltpu.make_async_remote_copy(
                src_ref=src,
                dst_ref=comm_ref.at[(j + 1) % 2],
                send_sem=send_sems.at[j % 2],
                recv_sem=recv_sems.at[(j + 1) % 2],
                device_id=(right,),
                device_id_type=pl.DeviceIdType.MESH,
            )
            if j >= 1:
                pl.semaphore_wait(credit_sem, 1)
            rdma.start()
            rdma.wait()
            if j < N_DEV - 2:
                pl.semaphore_signal(
                    credit_sem, inc=1,
                    device_id=(left,), device_id_type=pl.DeviceIdType.MESH,
                )
            origin = lax.rem(my - j - 1 + N_DEV, N_DEV)
            accumulate(origin, comm_ref[(j + 1) % 2], init=False)

    return pl.pallas_call(
        body,
        out_shape=jax.ShapeDtypeStruct((T, H), jnp.float32),
        in_specs=[pl.BlockSpec(memory_space=pltpu.VMEM)] * 4,
        out_specs=pl.BlockSpec(memory_space=pltpu.VMEM),
        scratch_shapes=[
            pltpu.VMEM((2, E_LOC, D, H), jnp.float32),
            pltpu.SemaphoreType.DMA((2,)),
            pltpu.SemaphoreType.DMA((2,)),
            pltpu.SemaphoreType.REGULAR,
        ],
        compiler_params=pltpu.CompilerParams(collective_id=0),
    )(x, router_W, route_idx, expert_W)


# device time: 128806 ns/iter; 3.3149x vs baseline; 3.3149x over previous
import sys

import jax
import jax.numpy as jnp
from jax import lax
from jax.experimental import pallas as pl
from jax.experimental.pallas import tpu as pltpu

N_DEV = 16
E_LOC = 4


def _build_ring():
    try:
        import distributed_mesh_v7x as dm

        mesh = dm.get_mesh("i", N_DEV)
        coords = [tuple(d.coords) for d in mesh.devices.flat]
        xs = sorted({c[0] for c in coords})
        ys = sorted({c[1] for c in coords})
        zs = sorted({c[2] for c in coords})
        if (
            len(set(coords)) == N_DEV
            and len(xs) == 2
            and len(ys) == 2
            and len(zs) == 4
        ):
            path = [
                (0, 0, 0), (1, 0, 0), (1, 0, 1), (1, 0, 2),
                (1, 0, 3), (1, 1, 3), (1, 1, 2), (1, 1, 1),
                (1, 1, 0), (0, 1, 0), (0, 1, 1), (0, 1, 2),
                (0, 1, 3), (0, 0, 3), (0, 0, 2), (0, 0, 1),
            ]
            idx = {c: i for i, c in enumerate(coords)}
            ring = [idx[(xs[a], ys[b], zs[c])] for a, b, c in path]
            if sorted(ring) == list(range(N_DEV)):
                return ring
    except Exception as e:
        print(f"_build_ring fallback: {type(e).__name__}: {e}", file=sys.stderr)
    return list(range(N_DEV))


RING = _build_ring()
INV_RING = [0] * N_DEV
for _p, _l in enumerate(RING):
    INV_RING[_l] = _p


def kernel(x, router_W, route_idx, expert_W):
    T, D = x.shape
    E = router_W.shape[1]
    H = expert_W.shape[2]
    N_R = 8
    N_L = 7

    def body(x_ref, rw_ref, idx_ref, ew_ref, ring_ref, inv_ref, out_ref,
             ewb_ref, commR, commL, sendR, recvR, sendL, recvL,
             credR, credL):
        my = lax.axis_index("i")
        i16 = lax.broadcasted_iota(jnp.int32, (1, N_DEV), 1)
        ring_t = ring_ref[:, :]
        inv_t = inv_ref[:, :]

        def lut(tbl, i):
            return jnp.sum(jnp.where(i16 == i, tbl, 0))

        r = lut(inv_t, my)
        right = lut(ring_t, lax.rem(r + 1, N_DEV))
        left = lut(ring_t, lax.rem(r + N_DEV - 1, N_DEV))

        barrier_sem = pltpu.get_barrier_semaphore()
        for nbr in (left, right):
            pl.semaphore_signal(
                barrier_sem, inc=1,
                device_id=(nbr,), device_id_type=pl.DeviceIdType.MESH,
            )
        pl.semaphore_wait(barrier_sem, 2)

        ewb_ref[:, :, :] = ew_ref[:, :, :].astype(jnp.bfloat16)

        xv = x_ref[:, :]
        xb = xv.astype(jnp.bfloat16)
        scores = jnp.dot(xv, rw_ref[:, :], preferred_element_type=jnp.float32)
        e0 = idx_ref[:, 0:1]
        e1 = idx_ref[:, 1:2]
        cols = lax.broadcasted_iota(jnp.int32, (T, E), 1)
        s0 = jnp.sum(jnp.where(cols == e0, scores, 0.0), axis=1, keepdims=True)
        s1 = jnp.sum(jnp.where(cols == e1, scores, 0.0), axis=1, keepdims=True)
        w0 = 1.0 / (1.0 + jnp.exp(s1 - s0))
        w1 = 1.0 - w0

        def accum(origin, w_block, init):
            contrib = None
            for k in range(E_LOC):
                e_id = origin * E_LOC + k
                col = (jnp.where(e0 == e_id, w0, 0.0)
                       + jnp.where(e1 == e_id, w1, 0.0))
                xs = xb * col.astype(jnp.bfloat16)
                part = jnp.dot(xs, w_block[k],
                               preferred_element_type=jnp.float32)
                contrib = part if contrib is None else contrib + part
            if init:
                out_ref[:, :] = contrib
            else:
                out_ref[:, :] += contrib

        for j in range(N_R + 1):
            send_r = j < N_R
            send_l = j < N_L
            rdma_r = rdma_l = None
            if send_r:
                rdma_r = pltpu.make_async_remote_copy(
                    src_ref=ewb_ref if j == 0 else commR.at[j % 2],
                    dst_ref=commR.at[(j + 1) % 2],
                    send_sem=sendR.at[j % 2],
                    recv_sem=recvR.at[(j + 1) % 2],
                    device_id=(right,),
                    device_id_type=pl.DeviceIdType.MESH,
                )
            if send_l:
                rdma_l = pltpu.make_async_remote_copy(
                    src_ref=ewb_ref if j == 0 else commL.at[j % 2],
                    dst_ref=commL.at[(j + 1) % 2],
                    send_sem=sendL.at[j % 2],
                    recv_sem=recvL.at[(j + 1) % 2],
                    device_id=(left,),
                    device_id_type=pl.DeviceIdType.MESH,
                )
            if send_r and j >= 1:
                pl.semaphore_wait(credR, 1)
            if send_l and j >= 1:
                pl.semaphore_wait(credL, 1)
            if rdma_r is not None:
                rdma_r.start()
            if rdma_l is not None:
                rdma_l.start()

            if j == 0:
                accum(my, ewb_ref[:, :, :], init=True)
            else:
                o_r = lut(ring_t, lax.rem(r - j + N_DEV, N_DEV))
                accum(o_r, commR[j % 2], init=False)
                if j <= N_L:
                    o_l = lut(ring_t, lax.rem(r + j, N_DEV))
                    accum(o_l, commL[j % 2], init=False)

            if rdma_r is not None:
                rdma_r.wait()
            if rdma_l is not None:
                rdma_l.wait()
            if send_r and j < N_R - 1:
                pl.semaphore_signal(
                    credR, inc=1,
                    device_id=(left,), device_id_type=pl.DeviceIdType.MESH,
                )
            if send_l and j < N_L - 1:
                pl.semaphore_signal(
                    credL, inc=1,
                    device_id=(right,), device_id_type=pl.DeviceIdType.MESH,
                )

    return pl.pallas_call(
        body,
        out_shape=jax.ShapeDtypeStruct((T, H), jnp.float32),
        in_specs=[pl.BlockSpec(memory_space=pltpu.VMEM)] * 6,
        out_specs=pl.BlockSpec(memory_space=pltpu.VMEM),
        scratch_shapes=[
            pltpu.VMEM((E_LOC, D, H), jnp.bfloat16),
            pltpu.VMEM((2, E_LOC, D, H), jnp.bfloat16),
            pltpu.VMEM((2, E_LOC, D, H), jnp.bfloat16),
            pltpu.SemaphoreType.DMA((2,)),
            pltpu.SemaphoreType.DMA((2,)),
            pltpu.SemaphoreType.DMA((2,)),
            pltpu.SemaphoreType.DMA((2,)),
            pltpu.SemaphoreType.REGULAR,
            pltpu.SemaphoreType.REGULAR,
        ],
        compiler_params=pltpu.CompilerParams(collective_id=0),
    )(x, router_W, route_idx, expert_W,
      jnp.asarray(RING, dtype=jnp.int32).reshape(1, N_DEV),
      jnp.asarray(INV_RING, dtype=jnp.int32).reshape(1, N_DEV))


# device time: 124188 ns/iter; 3.4382x vs baseline; 1.0372x over previous
import sys

import jax
import jax.numpy as jnp
from jax import lax
from jax.experimental import pallas as pl
from jax.experimental.pallas import tpu as pltpu

N_DEV = 16
E_LOC = 4
E_HALF = 2


def _build_ring():
    try:
        import distributed_mesh_v7x as dm

        mesh = dm.get_mesh("i", N_DEV)
        coords = [tuple(d.coords) for d in mesh.devices.flat]
        xs = sorted({c[0] for c in coords})
        ys = sorted({c[1] for c in coords})
        zs = sorted({c[2] for c in coords})
        if (
            len(set(coords)) == N_DEV
            and len(xs) == 2
            and len(ys) == 2
            and len(zs) == 4
        ):
            path = [
                (0, 0, 0), (1, 0, 0), (1, 0, 1), (1, 0, 2),
                (1, 0, 3), (1, 1, 3), (1, 1, 2), (1, 1, 1),
                (1, 1, 0), (0, 1, 0), (0, 1, 1), (0, 1, 2),
                (0, 1, 3), (0, 0, 3), (0, 0, 2), (0, 0, 1),
            ]
            idx = {c: i for i, c in enumerate(coords)}
            ring = [idx[(xs[a], ys[b], zs[c])] for a, b, c in path]
            if sorted(ring) == list(range(N_DEV)):
                return ring
    except Exception as e:
        print(f"_build_ring fallback: {type(e).__name__}: {e}", file=sys.stderr)
    return list(range(N_DEV))


RING = _build_ring()
INV_RING = [0] * N_DEV
for _p, _l in enumerate(RING):
    INV_RING[_l] = _p


def kernel(x, router_W, route_idx, expert_W):
    T, D = x.shape
    E = router_W.shape[1]
    H = expert_W.shape[2]
    N_R = 8
    N_L = 7

    def body(x_ref, rw_ref, idx_ref, ew_ref, ring_ref, inv_ref, out_ref,
             ewb_ref, bufs, sends, recvs, creds):
        my = lax.axis_index("i")
        i16 = lax.broadcasted_iota(jnp.int32, (1, N_DEV), 1)
        ring_t = ring_ref[:, :]
        inv_t = inv_ref[:, :]

        def lut(tbl, i):
            return jnp.sum(jnp.where(i16 == i, tbl, 0))

        r = lut(inv_t, my)
        right = lut(ring_t, lax.rem(r + 1, N_DEV))
        left = lut(ring_t, lax.rem(r + N_DEV - 1, N_DEV))

        barrier_sem = pltpu.get_barrier_semaphore()
        for nbr in (left, right):
            pl.semaphore_signal(
                barrier_sem, inc=1,
                device_id=(nbr,), device_id_type=pl.DeviceIdType.MESH,
            )
        pl.semaphore_wait(barrier_sem, 2)

        ewb_ref[:, :, :] = ew_ref[:, :, :].astype(jnp.bfloat16)

        xv = x_ref[:, :]
        xb = xv.astype(jnp.bfloat16)
        scores = jnp.dot(xv, rw_ref[:, :], preferred_element_type=jnp.float32)
        e0 = idx_ref[:, 0:1]
        e1 = idx_ref[:, 1:2]
        cols = lax.broadcasted_iota(jnp.int32, (T, E), 1)
        s0 = jnp.sum(jnp.where(cols == e0, scores, 0.0), axis=1, keepdims=True)
        s1 = jnp.sum(jnp.where(cols == e1, scores, 0.0), axis=1, keepdims=True)
        w0 = 1.0 / (1.0 + jnp.exp(s1 - s0))
        w1 = 1.0 - w0

        def accum(origin, w_half, k_base, init):
            contrib = None
            for k in range(E_HALF):
                e_id = origin * E_LOC + k_base + k
                col = (jnp.where(e0 == e_id, w0, 0.0)
                       + jnp.where(e1 == e_id, w1, 0.0))
                xs = xb * col.astype(jnp.bfloat16)
                part = jnp.dot(xs, w_half[k],
                               preferred_element_type=jnp.float32)
                contrib = part if contrib is None else contrib + part
            if init:
                out_ref[:, :] = contrib
            else:
                out_ref[:, :] += contrib

        SUBS = {("R", 0): 0, ("R", 1): 1, ("L", 0): 2, ("L", 1): 3}

        def make_rdma(dirn, half, j):
            s = SUBS[(dirn, half)]
            if j == 0:
                src = ewb_ref.at[pl.ds(half * E_HALF, E_HALF)]
            else:
                src = bufs.at[s, j % 2]
            return pltpu.make_async_remote_copy(
                src_ref=src,
                dst_ref=bufs.at[s, (j + 1) % 2],
                send_sem=sends.at[s, j % 2],
                recv_sem=recvs.at[s, (j + 1) % 2],
                device_id=(right if dirn == "R" else left,),
                device_id_type=pl.DeviceIdType.MESH,
            )

        for j in range(N_R + 1):
            send_r = j < N_R
            send_l = j < N_L
            o_r = lut(ring_t, lax.rem(r - j + N_DEV, N_DEV))
            o_l = lut(ring_t, lax.rem(r + j, N_DEV))
            rdmas = {}
            for half in (0, 1):
                for dirn, send in (("R", send_r), ("L", send_l)):
                    if not send:
                        continue
                    rd = make_rdma(dirn, half, j)
                    if j >= 1:
                        pl.semaphore_wait(creds.at[SUBS[(dirn, half)]], 1)
                    rd.start()
                    rdmas[(dirn, half)] = rd
                k_base = half * E_HALF
                if j == 0:
                    accum(my, ewb_ref[pl.ds(k_base, E_HALF)], k_base,
                          init=(half == 0))
                else:
                    accum(o_r, bufs[SUBS[("R", half)], j % 2], k_base,
                          init=False)
                    if j <= N_L:
                        accum(o_l, bufs[SUBS[("L", half)], j % 2], k_base,
                              init=False)
            for half in (0, 1):
                for dirn, send, n_dir, nbr in (
                    ("R", send_r, N_R, left), ("L", send_l, N_L, right)
                ):
                    if not send:
                        continue
                    rdmas[(dirn, half)].wait_send()
                    if j < n_dir - 1:
                        pl.semaphore_signal(
                            creds.at[SUBS[(dirn, half)]], inc=1,
                            device_id=(nbr,),
                            device_id_type=pl.DeviceIdType.MESH,
                        )
            for half in (0, 1):
                for dirn, send in (("R", send_r), ("L", send_l)):
                    if send:
                        rdmas[(dirn, half)].wait_recv()

    return pl.pallas_call(
        body,
        out_shape=jax.ShapeDtypeStruct((T, H), jnp.float32),
        in_specs=[pl.BlockSpec(memory_space=pltpu.VMEM)] * 6,
        out_specs=pl.BlockSpec(memory_space=pltpu.VMEM),
        scratch_shapes=[
            pltpu.VMEM((E_LOC, D, H), jnp.bfloat16),
            pltpu.VMEM((4, 2, E_HALF, D, H), jnp.bfloat16),
            pltpu.SemaphoreType.DMA((4, 2)),
            pltpu.SemaphoreType.DMA((4, 2)),
            pltpu.SemaphoreType.REGULAR((4,)),
        ],
        compiler_params=pltpu.CompilerParams(collective_id=0),
    )(x, router_W, route_idx, expert_W,
      jnp.asarray(RING, dtype=jnp.int32).reshape(1, N_DEV),
      jnp.asarray(INV_RING, dtype=jnp.int32).reshape(1, N_DEV))
